# baseline (speedup 1.0000x reference)
import functools
import jax
import jax.numpy as jnp
from jax.experimental import pallas as pl
from jax.experimental.pallas import tpu as pltpu

_BN_SCALE = 1.0 / jnp.sqrt(1.0 + 1e-5)


def _square_distance(src, dst):
    return (
        jnp.sum(src ** 2, -1)[..., :, None]
        + jnp.sum(dst ** 2, -1)[..., None, :]
        - 2.0 * jnp.einsum('bnc,bmc->bnm', src, dst)
    )


def _index_points(points, idx):
    B = points.shape[0]
    batch = jnp.arange(B).reshape((B,) + (1,) * (idx.ndim - 1))
    return points[batch, idx]


def _farthest_point_sample(xyz, npoint):
    B, N, _ = xyz.shape

    def step(carry, _):
        distance, farthest = carry
        centroid = jnp.take_along_axis(xyz, farthest[:, None, None], axis=1)
        dist = jnp.sum((xyz - centroid) ** 2, -1)
        distance = jnp.minimum(distance, dist)
        new_farthest = jnp.argmax(distance, -1).astype(jnp.int32)
        return (distance, new_farthest), farthest

    init = (jnp.full((B, N), 1e10, xyz.dtype), jnp.zeros((B,), jnp.int32))
    _, centroids = jax.lax.scan(step, init, None, length=npoint)
    return centroids.T


def _query_ball_point(radius, nsample, xyz, new_xyz):
    B, N, _ = xyz.shape
    S = new_xyz.shape[1]
    sqrdists = _square_distance(new_xyz, xyz)
    group_idx = jnp.broadcast_to(jnp.arange(N, dtype=jnp.int32), (B, S, N))
    group_idx = jnp.where(sqrdists > radius ** 2, N, group_idx)
    group_idx = jnp.sort(group_idx, axis=-1)[:, :, :nsample]
    group_first = jnp.broadcast_to(group_idx[:, :, :1], group_idx.shape)
    group_idx = jnp.where(group_idx == N, group_first, group_idx)
    return group_idx


def _bn_eval(x, gamma, beta):
    return x * _BN_SCALE * gamma + beta


def _set_abstraction(xyz, points, npoint, radius, nsample, layers):
    fps_idx = _farthest_point_sample(xyz, npoint)
    new_xyz = _index_points(xyz, fps_idx)
    idx = _query_ball_point(radius, nsample, xyz, new_xyz)
    grouped_xyz = _index_points(xyz, idx)
    grouped_xyz_norm = grouped_xyz - new_xyz[:, :, None, :]
    grouped_points = _index_points(points, idx)
    h = jnp.concatenate([grouped_xyz_norm, grouped_points], axis=-1)
    for w, b, g, be in layers:
        h = jnp.einsum('bskc,oc->bsko', h, w) + b
        h = jax.nn.relu(_bn_eval(h, g, be))
    return new_xyz, jnp.max(h, axis=2)


def _feature_propagation(xyz1, xyz2, points1, points2, layers):
    sqr = _square_distance(xyz1, xyz2)
    neg, idx = jax.lax.top_k(-sqr, 3)
    dists = -neg
    dist_recip = 1.0 / (dists + 1e-8)
    weight = dist_recip / jnp.sum(dist_recip, axis=2, keepdims=True)
    interpolated = jnp.sum(_index_points(points2, idx) * weight[..., None], axis=2)
    h = jnp.concatenate([points1, interpolated], axis=-1)
    for w, b, g, be in layers:
        h = jnp.einsum('bnc,oc->bno', h, w) + b
        h = jax.nn.relu(_bn_eval(h, g, be))
    return h


def _head_kernel(l0_ref, mean_ref, w1_ref, a1_ref, b1_ref, w2_ref, w2b_ref,
                 out_ref):
    x = l0_ref[...]
    m = mean_ref[...]
    w1 = w1_ref[...]
    C = x.shape[-1]
    h = jnp.dot(x[0], w1[:C], preferred_element_type=jnp.float32)
    h = h + jnp.dot(m[0], w1[C:], preferred_element_type=jnp.float32)
    h = h * a1_ref[...][0] + b1_ref[...][0]
    h = jnp.maximum(h, 0.0)
    o = jnp.dot(h, w2_ref[...], preferred_element_type=jnp.float32)
    out_ref[...] = (o + w2b_ref[...][0])[None]


def _head(l0_out, params):
    B, N, C = l0_out.shape
    w1, b1, g1, be1 = params['head_conv1']
    w2, b2 = params['head_conv2']
    a1 = (_BN_SCALE * g1)[None, :]
    bb1 = (b1 * _BN_SCALE * g1 + be1)[None, :]
    mean = jnp.mean(l0_out, axis=1, keepdims=True)
    TN = 1024
    out = pl.pallas_call(
        _head_kernel,
        grid=(B, N // TN),
        in_specs=[
            pl.BlockSpec((1, TN, C), lambda b, n: (b, n, 0)),
            pl.BlockSpec((1, 1, C), lambda b, n: (b, 0, 0)),
            pl.BlockSpec((2 * C, w1.shape[0]), lambda b, n: (0, 0)),
            pl.BlockSpec((1, w1.shape[0]), lambda b, n: (0, 0)),
            pl.BlockSpec((1, w1.shape[0]), lambda b, n: (0, 0)),
            pl.BlockSpec((w1.shape[0], w2.shape[0]), lambda b, n: (0, 0)),
            pl.BlockSpec((1, w2.shape[0]), lambda b, n: (0, 0)),
        ],
        out_specs=pl.BlockSpec((1, TN, w2.shape[0]), lambda b, n: (b, n, 0)),
        out_shape=jax.ShapeDtypeStruct((B, N, w2.shape[0]), jnp.float32),
    )(l0_out, mean, w1.T, a1, bb1, w2.T, b2[None, :])
    return out


@jax.jit
def _forward(coords, features, params):
    xyz0 = coords[..., :3]
    l0_points = features
    l1_xyz, l1_points = _set_abstraction(xyz0, l0_points, 1024, 0.5, 32,
                                         params['sa1'])
    l2_xyz, l2_points = _set_abstraction(l1_xyz, l1_points, 256, 1.0, 32,
                                         params['sa2'])
    l1_points = _feature_propagation(l1_xyz, l2_xyz, l1_points, l2_points,
                                     params['fp2'])
    l0_out = _feature_propagation(xyz0, l1_xyz, l0_points, l1_points,
                                  params['fp1'])
    log_odds = _head(l0_out, params)
    return jnp.concatenate([coords[..., :3], log_odds], axis=-1)


def kernel(coords, features, params):
    return _forward(coords, features, params)


# speedup vs baseline: 12.8957x; 12.8957x over previous
import functools
import numpy as np
import jax
import jax.numpy as jnp
from jax.experimental import pallas as pl
from jax.experimental.pallas import tpu as pltpu

_BN_SCALE = float(1.0 / np.sqrt(np.float32(1.0 + 1e-5)))


def _fps_body(npoint, x_ref, y_ref, z_ref, oi_ref, ox_ref, oy_ref, oz_ref):
    B, N = x_ref.shape
    x = x_ref[...]
    y = y_ref[...]
    z = z_ref[...]
    iota = jax.lax.broadcasted_iota(jnp.int32, (B, N), 1)

    def step(t, carry):
        dist, far = carry
        oi_ref[pl.ds(t, 1), :] = far.reshape(1, B)
        sel = iota == far[:, None]
        cx = jnp.sum(jnp.where(sel, x, 0.0), axis=1, keepdims=True)
        cy = jnp.sum(jnp.where(sel, y, 0.0), axis=1, keepdims=True)
        cz = jnp.sum(jnp.where(sel, z, 0.0), axis=1, keepdims=True)
        ox_ref[pl.ds(t, 1), :] = cx.reshape(1, B)
        oy_ref[pl.ds(t, 1), :] = cy.reshape(1, B)
        oz_ref[pl.ds(t, 1), :] = cz.reshape(1, B)
        d = (x - cx) ** 2 + (y - cy) ** 2 + (z - cz) ** 2
        dist = jnp.minimum(dist, d)
        mx = jnp.max(dist, axis=1, keepdims=True)
        far_new = jnp.min(jnp.where(dist == mx, iota, N), axis=1).astype(jnp.int32)
        return dist, far_new

    init = (jnp.full((B, N), 1e10, jnp.float32), jnp.zeros((B,), jnp.int32))
    jax.lax.fori_loop(0, npoint, step, init)


def _fps(x, y, z, npoint):
    B, N = x.shape
    outs = pl.pallas_call(
        functools.partial(_fps_body, npoint),
        grid=(1,),
        in_specs=[pl.BlockSpec((B, N), lambda i: (0, 0))] * 3,
        out_specs=[pl.BlockSpec((npoint, B), lambda i: (0, 0))] * 4,
        out_shape=[jax.ShapeDtypeStruct((npoint, B), jnp.int32)]
        + [jax.ShapeDtypeStruct((npoint, B), jnp.float32)] * 3,
    )(x, y, z)
    oi, ox, oy, oz = outs
    return oi.T, ox.T, oy.T, oz.T


def _proj_body(w_ref, x_ref, o_ref):
    w = w_ref[...]
    x = x_ref[0]
    o_ref[...] = jnp.dot(w, x, preferred_element_type=jnp.float32)[None]


def _proj(w, xT):
    B, C, N = xT.shape
    O = w.shape[0]
    return pl.pallas_call(
        _proj_body,
        grid=(B,),
        in_specs=[
            pl.BlockSpec((O, C), lambda b_: (0, 0)),
            pl.BlockSpec((1, C, N), lambda b_: (b_, 0, 0)),
        ],
        out_specs=pl.BlockSpec((1, O, N), lambda b_: (b_, 0, 0)),
        out_shape=jax.ShapeDtypeStruct((B, O, N), jnp.float32),
    )(w, xT)


def _gather_lanes(arr, idx):
    N = arr.shape[1]
    if N <= 128:
        return jnp.take_along_axis(arr, idx, axis=1)
    lane = jax.lax.bitwise_and(idx, 127)
    chunk = jax.lax.shift_right_logical(idx, 7)
    out = None
    for c in range(N // 128):
        g = jnp.take_along_axis(arr[:, c * 128:(c + 1) * 128], lane, axis=1)
        out = g if out is None else jnp.where(chunk == c, g, out)
    return out


def _sa_body(r2, nsample, layers23, x_ref, y_ref, z_ref, qxs_ref, qys_ref,
             qzs_ref, qxl_ref, qyl_ref, qzl_ref, p_ref, w1x_ref, w1y_ref,
             w1z_ref, bb1_ref, *wb_refs_and_out):
    *wb_refs, o_ref = wb_refs_and_out
    N = x_ref.shape[2]
    ST = qxs_ref.shape[1]
    C1 = p_ref.shape[1]
    n_chunks = N // 128
    n_bits = N.bit_length() - 1

    x = x_ref[0]
    y = y_ref[0]
    z = z_ref[0]
    qx = qxs_ref[0]
    qy = qys_ref[0]
    qz = qzs_ref[0]

    pp = x * x + y * y + z * z
    qq = qx * qx + qy * qy + qz * qz
    q3 = jnp.concatenate([qx, qy, qz], axis=1)
    p3 = jnp.concatenate([x, y, z], axis=0)
    cross = jnp.dot(q3, p3, preferred_element_type=jnp.float32)
    D = qq + pp - 2.0 * cross
    m = jnp.where(D <= r2, 1.0, 0.0)

    li = jax.lax.broadcasted_iota(jnp.int32, (128, 128), 0)
    lj = jax.lax.broadcasted_iota(jnp.int32, (128, 128), 1)
    U = jnp.where(li <= lj, 1.0, 0.0).astype(jnp.bfloat16)
    run = jnp.zeros((ST, 1), jnp.float32)
    cols = []
    for j in range(n_chunks):
        mc = m[:, j * 128:(j + 1) * 128].astype(jnp.bfloat16)
        cj = jnp.dot(mc, U, preferred_element_type=jnp.float32) + run
        cols.append(cj)
        run = cj[:, 127:128]
    C = jnp.concatenate(cols, axis=1)

    kk = (jax.lax.broadcasted_iota(jnp.int32, (ST, nsample), 1) + 1
          ).astype(jnp.float32)
    pos = jnp.zeros((ST, nsample), jnp.int32)
    for sb in range(n_bits, -1, -1):
        sh = 1 << sb
        cand = pos + sh
        cv = _gather_lanes(C, cand - 1)
        pos = jnp.where(cv < kk, cand, pos)
    total = C[:, N - 1:N]
    valid = kk <= total
    pos = jnp.where(valid, pos, pos[:, 0:1])
    posT = jnp.transpose(pos)

    qa_base = (w1x_ref[...] * qxl_ref[0]
               + w1y_ref[...] * qyl_ref[0]
               + w1z_ref[...] * qzl_ref[0])
    pre = bb1_ref[...] - qa_base

    Gs = []
    for k in range(nsample):
        rowk = posT[k:k + 1, :]
        Gs.append(_gather_lanes(p_ref[0], jnp.broadcast_to(rowk, (C1, ST))))
    G = jnp.concatenate(Gs, axis=1)
    pre_full = jnp.concatenate([pre] * nsample, axis=1)

    h = jnp.maximum(G + pre_full, 0.0)
    for wi in range(0, len(wb_refs), 2):
        w = wb_refs[wi][...]
        bcol = wb_refs[wi + 1][...]
        h = jnp.maximum(jnp.dot(w, h, preferred_element_type=jnp.float32)
                        + bcol, 0.0)

    L = ST * nsample
    hm = h
    sh = nsample // 2
    while sh >= 1:
        hm = jnp.maximum(hm, pltpu.roll(hm, L - sh * ST, axis=1))
        sh //= 2
    o_ref[...] = hm[:, :ST][None]


def _sa(xyzc, qc, qlc, pT, w1xyz_cols, bb1, layers23, r2, nsample, ST):
    B, C1, N = pT.shape
    S = qc[0].shape[1]
    Cout = layers23[-1][0].shape[0]
    grid = (B, S // ST)
    in_specs = (
        [pl.BlockSpec((1, 1, N), lambda b, s: (b, 0, 0))] * 3
        + [pl.BlockSpec((1, ST, 1), lambda b, s: (b, s, 0))] * 3
        + [pl.BlockSpec((1, 1, ST), lambda b, s: (b, 0, s))] * 3
        + [pl.BlockSpec((1, C1, N), lambda b, s: (b, 0, 0))]
        + [pl.BlockSpec((C1, 1), lambda b, s: (0, 0))] * 4
    )
    args = list(xyzc) + list(qc) + list(qlc) + [pT] + list(w1xyz_cols) + [bb1]
    for w, bcol in layers23:
        in_specs.append(pl.BlockSpec(w.shape, lambda b, s: (0, 0)))
        in_specs.append(pl.BlockSpec(bcol.shape, lambda b, s: (0, 0)))
        args += [w, bcol]
    return pl.pallas_call(
        functools.partial(_sa_body, r2, nsample, len(layers23)),
        grid=grid,
        in_specs=in_specs,
        out_specs=pl.BlockSpec((1, Cout, ST), lambda b, s: (b, 0, s)),
        out_shape=jax.ShapeDtypeStruct((B, Cout, S), jnp.float32),
    )(*args)


def _fp_body(layers, qxs_ref, qys_ref, qzs_ref, x2l_ref, y2l_ref, z2l_ref,
             p2_ref, p1_ref, *wb_and_out):
    *wb_refs, o_ref, msum_ref = wb_and_out
    N2 = x2l_ref.shape[2]
    NT = qxs_ref.shape[1]
    C2 = p2_ref.shape[1]

    qx = qxs_ref[0]
    qy = qys_ref[0]
    qz = qzs_ref[0]
    x2 = x2l_ref[0]
    y2 = y2l_ref[0]
    z2 = z2l_ref[0]

    qq = qx * qx + qy * qy + qz * qz
    pp = x2 * x2 + y2 * y2 + z2 * z2
    q3 = jnp.concatenate([qx, qy, qz], axis=1)
    p3 = jnp.concatenate([x2, y2, z2], axis=0)
    cross = jnp.dot(q3, p3, preferred_element_type=jnp.float32)
    D = qq + pp - 2.0 * cross

    li = jax.lax.broadcasted_iota(jnp.int32, (NT, N2), 1)
    ds = []
    js = []
    Dw = D
    for _ in range(3):
        dmin = jnp.min(Dw, axis=1, keepdims=True)
        jmin = jnp.min(jnp.where(Dw == dmin, li, N2), axis=1,
                       keepdims=True)
        ds.append(dmin)
        js.append(jmin)
        Dw = jnp.where(li == jmin, jnp.float32(np.inf), Dw)

    r1 = 1.0 / (ds[0] + 1e-8)
    r2_ = 1.0 / (ds[1] + 1e-8)
    r3 = 1.0 / (ds[2] + 1e-8)
    wsum = r1 + r2_ + r3
    w1T = jnp.transpose(r1 / wsum)
    w2T = jnp.transpose(r2_ / wsum)
    w3T = jnp.transpose(r3 / wsum)
    j1T = jnp.transpose(js[0])
    j2T = jnp.transpose(js[1])
    j3T = jnp.transpose(js[2])
    p2 = p2_ref[0]
    interp = (
        _gather_lanes(p2, jnp.broadcast_to(j1T, (C2, NT))) * w1T
        + _gather_lanes(p2, jnp.broadcast_to(j2T, (C2, NT))) * w2T
        + _gather_lanes(p2, jnp.broadcast_to(j3T, (C2, NT))) * w3T
    )
    h = jnp.concatenate([p1_ref[0], interp], axis=0)
    for wi in range(0, len(wb_refs), 2):
        w = wb_refs[wi][...]
        bcol = wb_refs[wi + 1][...]
        h = jnp.maximum(jnp.dot(w, h, preferred_element_type=jnp.float32)
                        + bcol, 0.0)
    o_ref[...] = h[None]

    part = jnp.sum(h, axis=1, keepdims=True)[None]
    @pl.when(pl.program_id(1) == 0)
    def _():
        msum_ref[...] = jnp.zeros_like(msum_ref)
    msum_ref[...] += part


def _fp(qs, x2l, p2T, p1T, layers, NT):
    B, C2, N2 = p2T.shape
    C1p = p1T.shape[1]
    N1 = qs[0].shape[1]
    Cout = layers[-1][0].shape[0]
    grid = (B, N1 // NT)
    in_specs = (
        [pl.BlockSpec((1, NT, 1), lambda b, n: (b, n, 0))] * 3
        + [pl.BlockSpec((1, 1, N2), lambda b, n: (b, 0, 0))] * 3
        + [pl.BlockSpec((1, C2, N2), lambda b, n: (b, 0, 0)),
           pl.BlockSpec((1, C1p, NT), lambda b, n: (b, 0, n))]
    )
    args = list(qs) + list(x2l) + [p2T, p1T]
    for w, bcol in layers:
        in_specs.append(pl.BlockSpec(w.shape, lambda b, n: (0, 0)))
        in_specs.append(pl.BlockSpec(bcol.shape, lambda b, n: (0, 0)))
        args += [w, bcol]
    outs = pl.pallas_call(
        functools.partial(_fp_body, len(layers)),
        grid=grid,
        in_specs=in_specs,
        out_specs=[
            pl.BlockSpec((1, Cout, NT), lambda b, n: (b, 0, n)),
            pl.BlockSpec((1, Cout, 1), lambda b, n: (b, 0, 0)),
        ],
        out_shape=[
            jax.ShapeDtypeStruct((B, Cout, N1), jnp.float32),
            jax.ShapeDtypeStruct((B, Cout, 1), jnp.float32),
        ],
    )(*args)
    return outs


def _head_body(n_total, x_ref, ms_ref, w1a_ref, w1b_ref, b1_ref, w2_ref,
               b2_ref, o_ref):
    x = x_ref[0]
    mean = ms_ref[0] * (1.0 / n_total)
    mc = jnp.dot(w1b_ref[...], mean, preferred_element_type=jnp.float32)
    h = jnp.dot(w1a_ref[...], x, preferred_element_type=jnp.float32)
    h = jnp.maximum(h + mc + b1_ref[...], 0.0)
    o = jnp.dot(w2_ref[...], h, preferred_element_type=jnp.float32)
    o_ref[...] = (o + b2_ref[...])[None]


def _head(l0T, msum, w1a, w1b, b1col, w2, b2col, NT):
    B, C, N = l0T.shape
    O1 = w1a.shape[0]
    return pl.pallas_call(
        functools.partial(_head_body, float(N)),
        grid=(B, N // NT),
        in_specs=[
            pl.BlockSpec((1, C, NT), lambda b, n: (b, 0, n)),
            pl.BlockSpec((1, C, 1), lambda b, n: (b, 0, 0)),
            pl.BlockSpec((O1, C), lambda b, n: (0, 0)),
            pl.BlockSpec((O1, C), lambda b, n: (0, 0)),
            pl.BlockSpec((O1, 1), lambda b, n: (0, 0)),
            pl.BlockSpec((1, O1), lambda b, n: (0, 0)),
            pl.BlockSpec((1, 1), lambda b, n: (0, 0)),
        ],
        out_specs=pl.BlockSpec((1, 1, NT), lambda b, n: (b, 0, n)),
        out_shape=jax.ShapeDtypeStruct((B, 1, N), jnp.float32),
    )(l0T, msum, w1a, w1b, b1col, w2, b2col)


def _fold(layers):
    out = []
    for w, b, g, be in layers:
        a = _BN_SCALE * g
        out.append((w * a[:, None], (b * a + be)[:, None]))
    return out


def _sqdist(src, dst):
    return (jnp.sum(src ** 2, -1)[..., :, None]
            + jnp.sum(dst ** 2, -1)[..., None, :]
            - 2.0 * jnp.einsum('bnc,bmc->bnm', src, dst))


def _jax_fp(xyz1, xyz2, points1, points2, layers):
    sqr = _sqdist(xyz1, xyz2)
    neg, idx = jax.lax.top_k(-sqr, 3)
    dists = -neg
    dist_recip = 1.0 / (dists + 1e-8)
    weight = dist_recip / jnp.sum(dist_recip, axis=2, keepdims=True)
    B = points2.shape[0]
    batch = jnp.arange(B).reshape((B,) + (1,) * (idx.ndim - 1))
    interpolated = jnp.sum(points2[batch, idx] * weight[..., None], axis=2)
    h = jnp.concatenate([points1, interpolated], axis=-1)
    for w, bcol in layers:
        h = jax.nn.relu(jnp.einsum('bnc,oc->bno', h, w) + bcol[:, 0])
    return h


@jax.jit
def _forward(coords, features, params):
    B, N, _ = coords.shape
    xyz = coords[..., :3]
    x0 = xyz[..., 0]
    y0 = xyz[..., 1]
    z0 = xyz[..., 2]

    sa1 = _fold(params['sa1'])
    w1, bb1 = sa1[0]
    _, qx1, qy1, qz1 = _fps(x0, y0, z0, 1024)
    x7T = jnp.concatenate([xyz.transpose(0, 2, 1),
                           features.transpose(0, 2, 1)], axis=1)
    p1T = _proj(w1, x7T)
    l1T = _sa(
        (x0[:, None], y0[:, None], z0[:, None]),
        (qx1[..., None], qy1[..., None], qz1[..., None]),
        (qx1[:, None, :], qy1[:, None, :], qz1[:, None, :]),
        p1T,
        (w1[:, 0:1], w1[:, 1:2], w1[:, 2:3]),
        bb1,
        sa1[1:],
        0.25, 32, 256,
    )

    sa2 = _fold(params['sa2'])
    w2, bb2 = sa2[0]
    _, qx2, qy2, qz2 = _fps(qx1, qy1, qz1, 256)
    x67T = jnp.concatenate([qx1[:, None, :], qy1[:, None, :],
                            qz1[:, None, :], l1T], axis=1)
    p2T = _proj(w2, x67T)
    l2T = _sa(
        (qx1[:, None], qy1[:, None], qz1[:, None]),
        (qx2[..., None], qy2[..., None], qz2[..., None]),
        (qx2[:, None, :], qy2[:, None, :], qz2[:, None, :]),
        p2T,
        (w2[:, 0:1], w2[:, 1:2], w2[:, 2:3]),
        bb2,
        sa2[1:],
        1.0, 32, 256,
    )

    fp2 = _fold(params['fp2'])
    l1pT, _ = _fp(
        (qx1[..., None], qy1[..., None], qz1[..., None]),
        (qx2[:, None, :], qy2[:, None, :], qz2[:, None, :]),
        l2T, l1T, fp2, 512,
    )

    fp1 = _fold(params['fp1'])
    l0T, msum = _fp(
        (x0[..., None], y0[..., None], z0[..., None]),
        (qx1[:, None, :], qy1[:, None, :], qz1[:, None, :]),
        l1pT, features.transpose(0, 2, 1), fp1, 1024,
    )

    w1h, b1h, g1h, be1h = params['head_conv1']
    a1 = _BN_SCALE * g1h
    w1f = w1h * a1[:, None]
    b1f = (b1h * a1 + be1h)[:, None]
    w2h, b2h = params['head_conv2']
    lo = _head(l0T, msum, w1f[:, :64], w1f[:, 64:], b1f, w2h, b2h[:, None],
               1024)
    return jnp.concatenate([xyz, lo.transpose(0, 2, 1)], axis=-1)


def kernel(coords, features, params):
    return _forward(coords, features, params)


# speedup vs baseline: 15.0396x; 1.1662x over previous
import functools
import numpy as np
import jax
import jax.numpy as jnp
from jax.experimental import pallas as pl
from jax.experimental.pallas import tpu as pltpu

_BN_SCALE = float(1.0 / np.sqrt(np.float32(1.0 + 1e-5)))


def _fps_body(npoint, x_ref, y_ref, z_ref, oi_ref, ox_ref, oy_ref, oz_ref):
    B, N = x_ref.shape
    x = x_ref[...]
    y = y_ref[...]
    z = z_ref[...]
    iota = jax.lax.broadcasted_iota(jnp.int32, (B, N), 1)

    def step(t, carry):
        dist, far = carry
        oi_ref[pl.ds(t, 1), :] = far.reshape(1, B)
        sel = iota == far[:, None]
        cx = jnp.sum(jnp.where(sel, x, 0.0), axis=1, keepdims=True)
        cy = jnp.sum(jnp.where(sel, y, 0.0), axis=1, keepdims=True)
        cz = jnp.sum(jnp.where(sel, z, 0.0), axis=1, keepdims=True)
        ox_ref[pl.ds(t, 1), :] = cx.reshape(1, B)
        oy_ref[pl.ds(t, 1), :] = cy.reshape(1, B)
        oz_ref[pl.ds(t, 1), :] = cz.reshape(1, B)
        d = (x - cx) ** 2 + (y - cy) ** 2 + (z - cz) ** 2
        dist = jnp.minimum(dist, d)
        mx = jnp.max(dist, axis=1, keepdims=True)
        far_new = jnp.min(jnp.where(dist == mx, iota, N), axis=1).astype(jnp.int32)
        return dist, far_new

    init = (jnp.full((B, N), 1e10, jnp.float32), jnp.zeros((B,), jnp.int32))
    jax.lax.fori_loop(0, npoint, step, init)


def _fps(x, y, z, npoint):
    B, N = x.shape
    outs = pl.pallas_call(
        functools.partial(_fps_body, npoint),
        grid=(1,),
        in_specs=[pl.BlockSpec((B, N), lambda i: (0, 0))] * 3,
        out_specs=[pl.BlockSpec((npoint, B), lambda i: (0, 0))] * 4,
        out_shape=[jax.ShapeDtypeStruct((npoint, B), jnp.int32)]
        + [jax.ShapeDtypeStruct((npoint, B), jnp.float32)] * 3,
    )(x, y, z)
    oi, ox, oy, oz = outs
    return oi.T, ox.T, oy.T, oz.T


def _proj_body(w_ref, x_ref, o_ref):
    w = w_ref[...]
    x = x_ref[0]
    o_ref[...] = jnp.dot(w, x, preferred_element_type=jnp.float32)[None]


def _proj(w, xT):
    B, C, N = xT.shape
    O = w.shape[0]
    return pl.pallas_call(
        _proj_body,
        grid=(B,),
        in_specs=[
            pl.BlockSpec((O, C), lambda b_: (0, 0)),
            pl.BlockSpec((1, C, N), lambda b_: (b_, 0, 0)),
        ],
        out_specs=pl.BlockSpec((1, O, N), lambda b_: (b_, 0, 0)),
        out_shape=jax.ShapeDtypeStruct((B, O, N), jnp.float32),
    )(w, xT)


def _gather_lanes(arr, idx):
    N = arr.shape[1]
    if N <= 128:
        return jnp.take_along_axis(arr, idx, axis=1)
    lane = jax.lax.bitwise_and(idx, 127)
    chunk = jax.lax.shift_right_logical(idx, 7)
    out = None
    for c in range(N // 128):
        g = jnp.take_along_axis(arr[:, c * 128:(c + 1) * 128], lane, axis=1)
        out = g if out is None else jnp.where(chunk == c, g, out)
    return out


def _sa_body(r2, nsample, gather_x, x_ref, y_ref, z_ref, qxs_ref, qys_ref,
             qzs_ref, qxl_ref, qyl_ref, qzl_ref, p_ref, w1x_ref, w1y_ref,
             w1z_ref, bb1_ref, *wb_refs_and_out):
    *wb_refs, o_ref = wb_refs_and_out
    N = x_ref.shape[2]
    ST = qxs_ref.shape[1]
    C1 = p_ref.shape[1]
    n_chunks = N // 128
    n_bits = N.bit_length() - 1

    x = x_ref[0]
    y = y_ref[0]
    z = z_ref[0]
    qx = qxs_ref[0]
    qy = qys_ref[0]
    qz = qzs_ref[0]

    pp = x * x + y * y + z * z
    qq = qx * qx + qy * qy + qz * qz
    q3 = jnp.concatenate([qx, qy, qz], axis=1)
    p3 = jnp.concatenate([x, y, z], axis=0)
    cross = jnp.dot(q3, p3, preferred_element_type=jnp.float32)
    D = qq + pp - 2.0 * cross
    m = jnp.where(D <= r2, 1.0, 0.0)

    li = jax.lax.broadcasted_iota(jnp.int32, (128, 128), 0)
    lj = jax.lax.broadcasted_iota(jnp.int32, (128, 128), 1)
    U = jnp.where(li <= lj, 1.0, 0.0).astype(jnp.bfloat16)
    run = jnp.zeros((ST, 1), jnp.float32)
    cols = []
    for j in range(n_chunks):
        mc = m[:, j * 128:(j + 1) * 128].astype(jnp.bfloat16)
        cj = jnp.dot(mc, U, preferred_element_type=jnp.float32) + run
        cols.append(cj)
        run = cj[:, 127:128]
    C = jnp.concatenate(cols, axis=1)

    kk = (jax.lax.broadcasted_iota(jnp.int32, (ST, nsample), 1) + 1
          ).astype(jnp.float32)
    pos = jnp.zeros((ST, nsample), jnp.int32)
    for sb in range(n_bits, -1, -1):
        sh = 1 << sb
        cand = pos + sh
        cv = _gather_lanes(C, cand - 1)
        pos = jnp.where(cv < kk, cand, pos)
    total = C[:, N - 1:N]
    valid = kk <= total
    pos = jnp.where(valid, pos, pos[:, 0:1])
    posT = jnp.transpose(pos)

    Gs = []
    for k in range(nsample):
        rowk = posT[k:k + 1, :]
        Gs.append(_gather_lanes(p_ref[0], jnp.broadcast_to(rowk, (C1, ST))))
    G = jnp.concatenate(Gs, axis=1)

    if gather_x:
        qxt = jnp.concatenate([qxl_ref[0]] * nsample, axis=1)
        qyt = jnp.concatenate([qyl_ref[0]] * nsample, axis=1)
        qzt = jnp.concatenate([qzl_ref[0]] * nsample, axis=1)
        Gc = jnp.concatenate(
            [G[0:1] - qxt, G[1:2] - qyt, G[2:3] - qzt, G[3:]], axis=0)
        w1 = w1x_ref[...]
        h = jnp.maximum(
            jnp.dot(w1, Gc, preferred_element_type=jnp.float32)
            + bb1_ref[...], 0.0)
    else:
        qa_base = (w1x_ref[...] * qxl_ref[0]
                   + w1y_ref[...] * qyl_ref[0]
                   + w1z_ref[...] * qzl_ref[0])
        pre = bb1_ref[...] - qa_base
        pre_full = jnp.concatenate([pre] * nsample, axis=1)
        h = jnp.maximum(G + pre_full, 0.0)
    for wi in range(0, len(wb_refs), 2):
        w = wb_refs[wi][...]
        bcol = wb_refs[wi + 1][...]
        h = jnp.maximum(jnp.dot(w, h, preferred_element_type=jnp.float32)
                        + bcol, 0.0)

    L = ST * nsample
    hm = h
    sh = nsample // 2
    while sh >= 1:
        hm = jnp.maximum(hm, pltpu.roll(hm, L - sh * ST, axis=1))
        sh //= 2
    o_ref[...] = hm[:, :ST][None]


def _sa(xyzc, qc, qlc, pT, w1xyz_cols, bb1, layers23, r2, nsample, ST,
        gather_x=False):
    B, C1, N = pT.shape
    S = qc[0].shape[1]
    Cout = layers23[-1][0].shape[0]
    grid = (B, S // ST)
    in_specs = (
        [pl.BlockSpec((1, 1, N), lambda b, s: (b, 0, 0))] * 3
        + [pl.BlockSpec((1, ST, 1), lambda b, s: (b, s, 0))] * 3
        + [pl.BlockSpec((1, 1, ST), lambda b, s: (b, 0, s))] * 3
        + [pl.BlockSpec((1, C1, N), lambda b, s: (b, 0, 0))]
        + [pl.BlockSpec(a.shape, lambda b, s: (0, 0))
           for a in list(w1xyz_cols) + [bb1]]
    )
    args = list(xyzc) + list(qc) + list(qlc) + [pT] + list(w1xyz_cols) + [bb1]
    for w, bcol in layers23:
        in_specs.append(pl.BlockSpec(w.shape, lambda b, s: (0, 0)))
        in_specs.append(pl.BlockSpec(bcol.shape, lambda b, s: (0, 0)))
        args += [w, bcol]
    return pl.pallas_call(
        functools.partial(_sa_body, r2, nsample, gather_x),
        grid=grid,
        in_specs=in_specs,
        out_specs=pl.BlockSpec((1, Cout, ST), lambda b, s: (b, 0, s)),
        out_shape=jax.ShapeDtypeStruct((B, Cout, S), jnp.float32),
    )(*args)


def _fp_body(layers, qxs_ref, qys_ref, qzs_ref, x2l_ref, y2l_ref, z2l_ref,
             p2_ref, p1_ref, *wb_and_out):
    *wb_refs, o_ref, msum_ref = wb_and_out
    N2 = x2l_ref.shape[2]
    NT = qxs_ref.shape[1]
    C2 = p2_ref.shape[1]

    qx = qxs_ref[0]
    qy = qys_ref[0]
    qz = qzs_ref[0]
    x2 = x2l_ref[0]
    y2 = y2l_ref[0]
    z2 = z2l_ref[0]

    qq = qx * qx + qy * qy + qz * qz
    pp = x2 * x2 + y2 * y2 + z2 * z2
    q3 = jnp.concatenate([qx, qy, qz], axis=1)
    p3 = jnp.concatenate([x2, y2, z2], axis=0)
    cross = jnp.dot(q3, p3, preferred_element_type=jnp.float32)
    D = qq + pp - 2.0 * cross

    li = jax.lax.broadcasted_iota(jnp.int32, (NT, N2), 1)
    ds = []
    js = []
    Dw = D
    for _ in range(3):
        dmin = jnp.min(Dw, axis=1, keepdims=True)
        jmin = jnp.min(jnp.where(Dw == dmin, li, N2), axis=1,
                       keepdims=True)
        ds.append(dmin)
        js.append(jmin)
        Dw = jnp.where(li == jmin, jnp.float32(np.inf), Dw)

    r1 = 1.0 / (ds[0] + 1e-8)
    r2_ = 1.0 / (ds[1] + 1e-8)
    r3 = 1.0 / (ds[2] + 1e-8)
    wsum = r1 + r2_ + r3
    w1T = jnp.transpose(r1 / wsum)
    w2T = jnp.transpose(r2_ / wsum)
    w3T = jnp.transpose(r3 / wsum)
    j1T = jnp.transpose(js[0])
    j2T = jnp.transpose(js[1])
    j3T = jnp.transpose(js[2])
    p2 = p2_ref[0]
    interp = (
        _gather_lanes(p2, jnp.broadcast_to(j1T, (C2, NT))) * w1T
        + _gather_lanes(p2, jnp.broadcast_to(j2T, (C2, NT))) * w2T
        + _gather_lanes(p2, jnp.broadcast_to(j3T, (C2, NT))) * w3T
    )
    h = jnp.concatenate([p1_ref[0], interp], axis=0)
    for wi in range(0, len(wb_refs), 2):
        w = wb_refs[wi][...]
        bcol = wb_refs[wi + 1][...]
        h = jnp.maximum(jnp.dot(w, h, preferred_element_type=jnp.float32)
                        + bcol, 0.0)
    o_ref[...] = h[None]

    part = jnp.sum(h, axis=1, keepdims=True)[None]
    @pl.when(pl.program_id(1) == 0)
    def _():
        msum_ref[...] = jnp.zeros_like(msum_ref)
    msum_ref[...] += part


def _fp(qs, x2l, p2T, p1T, layers, NT):
    B, C2, N2 = p2T.shape
    C1p = p1T.shape[1]
    N1 = qs[0].shape[1]
    Cout = layers[-1][0].shape[0]
    grid = (B, N1 // NT)
    in_specs = (
        [pl.BlockSpec((1, NT, 1), lambda b, n: (b, n, 0))] * 3
        + [pl.BlockSpec((1, 1, N2), lambda b, n: (b, 0, 0))] * 3
        + [pl.BlockSpec((1, C2, N2), lambda b, n: (b, 0, 0)),
           pl.BlockSpec((1, C1p, NT), lambda b, n: (b, 0, n))]
    )
    args = list(qs) + list(x2l) + [p2T, p1T]
    for w, bcol in layers:
        in_specs.append(pl.BlockSpec(w.shape, lambda b, n: (0, 0)))
        in_specs.append(pl.BlockSpec(bcol.shape, lambda b, n: (0, 0)))
        args += [w, bcol]
    outs = pl.pallas_call(
        functools.partial(_fp_body, len(layers)),
        grid=grid,
        in_specs=in_specs,
        out_specs=[
            pl.BlockSpec((1, Cout, NT), lambda b, n: (b, 0, n)),
            pl.BlockSpec((1, Cout, 1), lambda b, n: (b, 0, 0)),
        ],
        out_shape=[
            jax.ShapeDtypeStruct((B, Cout, N1), jnp.float32),
            jax.ShapeDtypeStruct((B, Cout, 1), jnp.float32),
        ],
    )(*args)
    return outs


def _head_body(n_total, x_ref, ms_ref, w1a_ref, w1b_ref, b1_ref, w2_ref,
               b2_ref, o_ref):
    x = x_ref[0]
    mean = ms_ref[0] * (1.0 / n_total)
    mc = jnp.dot(w1b_ref[...], mean, preferred_element_type=jnp.float32)
    h = jnp.dot(w1a_ref[...], x, preferred_element_type=jnp.float32)
    h = jnp.maximum(h + mc + b1_ref[...], 0.0)
    o = jnp.dot(w2_ref[...], h, preferred_element_type=jnp.float32)
    o_ref[...] = (o + b2_ref[...])[None]


def _head(l0T, msum, w1a, w1b, b1col, w2, b2col, NT):
    B, C, N = l0T.shape
    O1 = w1a.shape[0]
    return pl.pallas_call(
        functools.partial(_head_body, float(N)),
        grid=(B, N // NT),
        in_specs=[
            pl.BlockSpec((1, C, NT), lambda b, n: (b, 0, n)),
            pl.BlockSpec((1, C, 1), lambda b, n: (b, 0, 0)),
            pl.BlockSpec((O1, C), lambda b, n: (0, 0)),
            pl.BlockSpec((O1, C), lambda b, n: (0, 0)),
            pl.BlockSpec((O1, 1), lambda b, n: (0, 0)),
            pl.BlockSpec((1, O1), lambda b, n: (0, 0)),
            pl.BlockSpec((1, 1), lambda b, n: (0, 0)),
        ],
        out_specs=pl.BlockSpec((1, 1, NT), lambda b, n: (b, 0, n)),
        out_shape=jax.ShapeDtypeStruct((B, 1, N), jnp.float32),
    )(l0T, msum, w1a, w1b, b1col, w2, b2col)


def _fold(layers):
    out = []
    for w, b, g, be in layers:
        a = _BN_SCALE * g
        out.append((w * a[:, None], (b * a + be)[:, None]))
    return out


def _sqdist(src, dst):
    return (jnp.sum(src ** 2, -1)[..., :, None]
            + jnp.sum(dst ** 2, -1)[..., None, :]
            - 2.0 * jnp.einsum('bnc,bmc->bnm', src, dst))


def _jax_fp(xyz1, xyz2, points1, points2, layers):
    sqr = _sqdist(xyz1, xyz2)
    neg, idx = jax.lax.top_k(-sqr, 3)
    dists = -neg
    dist_recip = 1.0 / (dists + 1e-8)
    weight = dist_recip / jnp.sum(dist_recip, axis=2, keepdims=True)
    B = points2.shape[0]
    batch = jnp.arange(B).reshape((B,) + (1,) * (idx.ndim - 1))
    interpolated = jnp.sum(points2[batch, idx] * weight[..., None], axis=2)
    h = jnp.concatenate([points1, interpolated], axis=-1)
    for w, bcol in layers:
        h = jax.nn.relu(jnp.einsum('bnc,oc->bno', h, w) + bcol[:, 0])
    return h


@jax.jit
def _forward(coords, features, params):
    B, N, _ = coords.shape
    xyz = coords[..., :3]
    x0 = xyz[..., 0]
    y0 = xyz[..., 1]
    z0 = xyz[..., 2]

    sa1 = _fold(params['sa1'])
    w1, bb1 = sa1[0]
    _, qx1, qy1, qz1 = _fps(x0, y0, z0, 1024)
    x7T = jnp.concatenate([xyz.transpose(0, 2, 1),
                           features.transpose(0, 2, 1),
                           jnp.zeros((B, 1, N), jnp.float32)], axis=1)
    w1pad = jnp.concatenate([w1, jnp.zeros((w1.shape[0], 1), jnp.float32)],
                            axis=1)
    l1T = _sa(
        (x0[:, None], y0[:, None], z0[:, None]),
        (qx1[..., None], qy1[..., None], qz1[..., None]),
        (qx1[:, None, :], qy1[:, None, :], qz1[:, None, :]),
        x7T,
        (w1pad, w1[:, 0:1], w1[:, 1:2]),
        bb1,
        sa1[1:],
        0.25, 32, 256,
        gather_x=True,
    )

    sa2 = _fold(params['sa2'])
    w2, bb2 = sa2[0]
    _, qx2, qy2, qz2 = _fps(qx1, qy1, qz1, 256)
    x67T = jnp.concatenate([qx1[:, None, :], qy1[:, None, :],
                            qz1[:, None, :], l1T], axis=1)
    p2T = _proj(w2, x67T)
    l2T = _sa(
        (qx1[:, None], qy1[:, None], qz1[:, None]),
        (qx2[..., None], qy2[..., None], qz2[..., None]),
        (qx2[:, None, :], qy2[:, None, :], qz2[:, None, :]),
        p2T,
        (w2[:, 0:1], w2[:, 1:2], w2[:, 2:3]),
        bb2,
        sa2[1:],
        1.0, 32, 256,
    )

    fp2 = _fold(params['fp2'])
    l1pT, _ = _fp(
        (qx1[..., None], qy1[..., None], qz1[..., None]),
        (qx2[:, None, :], qy2[:, None, :], qz2[:, None, :]),
        l2T, l1T, fp2, 512,
    )

    fp1 = _fold(params['fp1'])
    l0T, msum = _fp(
        (x0[..., None], y0[..., None], z0[..., None]),
        (qx1[:, None, :], qy1[:, None, :], qz1[:, None, :]),
        l1pT, features.transpose(0, 2, 1), fp1, 1024,
    )

    w1h, b1h, g1h, be1h = params['head_conv1']
    a1 = _BN_SCALE * g1h
    w1f = w1h * a1[:, None]
    b1f = (b1h * a1 + be1h)[:, None]
    w2h, b2h = params['head_conv2']
    lo = _head(l0T, msum, w1f[:, :64], w1f[:, 64:], b1f, w2h, b2h[:, None],
               1024)
    return jnp.concatenate([xyz, lo.transpose(0, 2, 1)], axis=-1)


def kernel(coords, features, params):
    return _forward(coords, features, params)
